# baseline (speedup 1.0000x reference)
import functools

import jax
import jax.numpy as jnp
from jax import lax
from jax.experimental import pallas as pl
from jax.experimental.pallas import tpu as pltpu

_N = 4096
_NFEAT = 256
_NHID = 16
_NHEADS = 4
_ALPHA = 0.2
_BLK = 256


def _wh_body(x_ref, w_ref, wh_ref):
    wh = jnp.dot(x_ref[...], w_ref[0], preferred_element_type=jnp.float32)
    wh_ref[0] = jnp.concatenate(
        [wh, jnp.ones((_N, 1), jnp.float32)], axis=1)


def _gat_body(adj_ref, wh_ref, a_ref, out_ref):
    i = pl.program_id(0)
    adj = adj_ref[...].astype(jnp.bfloat16)
    for h in range(_NHEADS):
        wh = wh_ref[h]
        whb = wh_ref[h, pl.ds(i * _BLK, _BLK), :_NHID]
        a1 = a_ref[h, :_NHID, :]
        a2 = a_ref[h, _NHID:, :]
        f1 = lax.dot_general(whb, a1, (((1,), (0,)), ((), ())),
                             preferred_element_type=jnp.float32)
        f2 = lax.dot_general(a2, wh[:, :_NHID], (((0,), (1,)), ((), ())),
                             preferred_element_type=jnp.float32)
        m_raw = f1 + jnp.max(f2)
        m = jnp.maximum(m_raw, _ALPHA * m_raw)
        l2e = 1.4426950408889634
        g1 = ((f1 - m) * l2e).astype(jnp.bfloat16)
        g2 = ((_ALPHA * f1 - m) * l2e).astype(jnp.bfloat16)
        f2b = (f2 * l2e).astype(jnp.bfloat16)
        f2c = (f2 * (_ALPHA * l2e)).astype(jnp.bfloat16)
        y = jnp.maximum(g1 + f2b, g2 + f2c)
        p = jnp.exp2(y) * adj
        hp_aug = jnp.dot(p, wh.astype(jnp.bfloat16),
                         preferred_element_type=jnp.float32)
        hp = hp_aug[:, :_NHID] / hp_aug[:, _NHID:]
        out_ref[:, h * _NHID:(h + 1) * _NHID] = jnp.where(
            hp > 0.0, hp, jnp.exp(hp) - 1.0)


@jax.jit
def kernel(x, adj, Ws, As):
    wh = pl.pallas_call(
        _wh_body,
        grid=(_NHEADS,),
        in_specs=[
            pl.BlockSpec((_N, _NFEAT), lambda h: (0, 0)),
            pl.BlockSpec((1, _NFEAT, _NHID), lambda h: (h, 0, 0)),
        ],
        out_specs=pl.BlockSpec((1, _N, _NHID + 1), lambda h: (h, 0, 0)),
        out_shape=jax.ShapeDtypeStruct((_NHEADS, _N, _NHID + 1), jnp.float32),
    )(x, Ws)

    out = pl.pallas_call(
        _gat_body,
        grid=(_N // _BLK,),
        in_specs=[
            pl.BlockSpec((_BLK, _N), lambda i: (i, 0)),
            pl.BlockSpec((_NHEADS, _N, _NHID + 1), lambda i: (0, 0, 0)),
            pl.BlockSpec((_NHEADS, 2 * _NHID, 1), lambda i: (0, 0, 0)),
        ],
        out_specs=pl.BlockSpec((_BLK, _NHEADS * _NHID), lambda i: (i, 0)),
        out_shape=jax.ShapeDtypeStruct((_N, _NHEADS * _NHID), jnp.float32),
    )(adj, wh, As)
    return out


# speedup vs baseline: 1.2104x; 1.2104x over previous
import functools

import jax
import jax.numpy as jnp
from jax import lax
from jax.experimental import pallas as pl
from jax.experimental.pallas import tpu as pltpu

_N = 4096
_NFEAT = 256
_NHID = 16
_NHEADS = 4
_ALPHA = 0.2
_BLK = 1024


def _wh_body(x_ref, w_ref, wh_ref):
    wh = jnp.dot(x_ref[...], w_ref[0], preferred_element_type=jnp.float32)
    wh_ref[0] = jnp.concatenate(
        [wh, jnp.ones((_N, 1), jnp.float32)], axis=1)


def _gat_body(adj_ref, wh_ref, a_ref, out_ref):
    i = pl.program_id(0)
    adj = adj_ref[...].astype(jnp.bfloat16)
    for h in range(_NHEADS):
        wh = wh_ref[h]
        whb = wh_ref[h, pl.ds(i * _BLK, _BLK), :_NHID]
        a1 = a_ref[h, :_NHID, :]
        a2 = a_ref[h, _NHID:, :]
        f1 = lax.dot_general(whb, a1, (((1,), (0,)), ((), ())),
                             preferred_element_type=jnp.float32)
        f2 = lax.dot_general(a2, wh[:, :_NHID], (((0,), (1,)), ((), ())),
                             preferred_element_type=jnp.float32)
        m_raw = f1 + jnp.max(f2)
        m = jnp.maximum(m_raw, _ALPHA * m_raw)
        l2e = 1.4426950408889634
        g1 = ((f1 - m) * l2e).astype(jnp.bfloat16)
        g2 = ((_ALPHA * f1 - m) * l2e).astype(jnp.bfloat16)
        f2b = (f2 * l2e).astype(jnp.bfloat16)
        f2c = (f2 * (_ALPHA * l2e)).astype(jnp.bfloat16)
        y = jnp.maximum(g1 + f2b, g2 + f2c)
        p = jnp.exp2(y) * adj
        hp_aug = jnp.dot(p, wh.astype(jnp.bfloat16),
                         preferred_element_type=jnp.float32)
        hp = hp_aug[:, :_NHID] / hp_aug[:, _NHID:]
        out_ref[:, h * _NHID:(h + 1) * _NHID] = jnp.where(
            hp > 0.0, hp, jnp.exp(hp) - 1.0)


@jax.jit
def kernel(x, adj, Ws, As):
    wh = pl.pallas_call(
        _wh_body,
        grid=(_NHEADS,),
        in_specs=[
            pl.BlockSpec((_N, _NFEAT), lambda h: (0, 0)),
            pl.BlockSpec((1, _NFEAT, _NHID), lambda h: (h, 0, 0)),
        ],
        out_specs=pl.BlockSpec((1, _N, _NHID + 1), lambda h: (h, 0, 0)),
        out_shape=jax.ShapeDtypeStruct((_NHEADS, _N, _NHID + 1), jnp.float32),
    )(x, Ws)

    out = pl.pallas_call(
        _gat_body,
        grid=(_N // _BLK,),
        in_specs=[
            pl.BlockSpec((_BLK, _N), lambda i: (i, 0)),
            pl.BlockSpec((_NHEADS, _N, _NHID + 1), lambda i: (0, 0, 0)),
            pl.BlockSpec((_NHEADS, 2 * _NHID, 1), lambda i: (0, 0, 0)),
        ],
        out_specs=pl.BlockSpec((_BLK, _NHEADS * _NHID), lambda i: (i, 0)),
        out_shape=jax.ShapeDtypeStruct((_N, _NHEADS * _NHID), jnp.float32),
    )(adj, wh, As)
    return out


# speedup vs baseline: 1.3502x; 1.1155x over previous
import functools

import jax
import jax.numpy as jnp
from jax import lax
from jax.experimental import pallas as pl
from jax.experimental.pallas import tpu as pltpu

_N = 4096
_NFEAT = 256
_NHID = 16
_NHEADS = 4
_ALPHA = 0.2
_BLK = 512
_NBLK = _N // _BLK


def _wh_body(x_ref, w_ref, wh_ref):
    wh = jnp.dot(x_ref[...], w_ref[0], preferred_element_type=jnp.float32)
    wh_ref[0] = jnp.concatenate(
        [wh, jnp.ones((_N, 1), jnp.float32)], axis=1)


def _copy(adj_hbm, buf_ref, sem, blk, slot):
    return pltpu.make_async_copy(
        adj_hbm.at[pl.ds(blk * _BLK, _BLK), :], buf_ref.at[slot], sem.at[slot])


def _gat_body(adj_hbm, wh_ref, a_ref, out_ref, buf_ref, sem):
    i = pl.program_id(0)
    slot = lax.rem(i, 2)

    @pl.when(i == 0)
    def _():
        _copy(adj_hbm, buf_ref, sem, 0, 0).start()

    _copy(adj_hbm, buf_ref, sem, i, slot).wait()

    @pl.when(i + 1 < _NBLK)
    def _():
        _copy(adj_hbm, buf_ref, sem, i + 1, 1 - slot).start()

    adj = buf_ref[slot].astype(jnp.bfloat16)
    for h in range(_NHEADS):
        wh = wh_ref[h]
        whb = wh_ref[h, pl.ds(i * _BLK, _BLK), :_NHID]
        a1 = a_ref[h, :_NHID, :]
        a2 = a_ref[h, _NHID:, :]
        f1 = lax.dot_general(whb, a1, (((1,), (0,)), ((), ())),
                             preferred_element_type=jnp.float32)
        f2 = lax.dot_general(a2, wh[:, :_NHID], (((0,), (1,)), ((), ())),
                             preferred_element_type=jnp.float32)
        m_raw = f1 + jnp.max(f2)
        m = jnp.maximum(m_raw, _ALPHA * m_raw)
        l2e = 1.4426950408889634
        g1 = ((f1 - m) * l2e).astype(jnp.bfloat16)
        g2 = ((_ALPHA * f1 - m) * l2e).astype(jnp.bfloat16)
        f2b = (f2 * l2e).astype(jnp.bfloat16)
        f2c = (f2 * (_ALPHA * l2e)).astype(jnp.bfloat16)
        y = jnp.maximum(g1 + f2b, g2 + f2c)
        p = jnp.exp2(y) * adj
        hp_aug = jnp.dot(p, wh.astype(jnp.bfloat16),
                         preferred_element_type=jnp.float32)
        hp = hp_aug[:, :_NHID] / hp_aug[:, _NHID:]
        out_ref[:, h * _NHID:(h + 1) * _NHID] = jnp.where(
            hp > 0.0, hp, jnp.exp(hp) - 1.0)


@jax.jit
def kernel(x, adj, Ws, As):
    wh = pl.pallas_call(
        _wh_body,
        grid=(_NHEADS,),
        in_specs=[
            pl.BlockSpec((_N, _NFEAT), lambda h: (0, 0)),
            pl.BlockSpec((1, _NFEAT, _NHID), lambda h: (h, 0, 0)),
        ],
        out_specs=pl.BlockSpec((1, _N, _NHID + 1), lambda h: (h, 0, 0)),
        out_shape=jax.ShapeDtypeStruct((_NHEADS, _N, _NHID + 1), jnp.float32),
    )(x, Ws)

    out = pl.pallas_call(
        _gat_body,
        grid=(_NBLK,),
        in_specs=[
            pl.BlockSpec(memory_space=pltpu.MemorySpace.HBM),
            pl.BlockSpec((_NHEADS, _N, _NHID + 1), lambda i: (0, 0, 0)),
            pl.BlockSpec((_NHEADS, 2 * _NHID, 1), lambda i: (0, 0, 0)),
        ],
        out_specs=pl.BlockSpec((_BLK, _NHEADS * _NHID), lambda i: (i, 0)),
        out_shape=jax.ShapeDtypeStruct((_N, _NHEADS * _NHID), jnp.float32),
        scratch_shapes=[
            pltpu.VMEM((2, _BLK, _N), jnp.float32),
            pltpu.SemaphoreType.DMA((2,)),
        ],
    )(adj, wh, As)
    return out


# speedup vs baseline: 1.3707x; 1.0151x over previous
import functools

import jax
import jax.numpy as jnp
from jax import lax
from jax.experimental import pallas as pl
from jax.experimental.pallas import tpu as pltpu

_N = 4096
_NFEAT = 256
_NHID = 16
_NHEADS = 4
_ALPHA = 0.2
_BLK = 512


def _wh_body(x_ref, w_ref, wh_ref):
    wh = jnp.dot(x_ref[...], w_ref[0], preferred_element_type=jnp.float32)
    wh_ref[0] = jnp.concatenate(
        [wh, jnp.ones((_N, 1), jnp.float32)], axis=1)


def _gat_body(adj_ref, wh_ref, a_ref, out_ref):
    i = pl.program_id(0)
    adj = adj_ref[...].astype(jnp.bfloat16)
    for h in range(_NHEADS):
        wh = wh_ref[h]
        whb = wh_ref[h, pl.ds(i * _BLK, _BLK), :_NHID]
        a1 = a_ref[h, :_NHID, :]
        a2 = a_ref[h, _NHID:, :]
        f1 = lax.dot_general(whb, a1, (((1,), (0,)), ((), ())),
                             preferred_element_type=jnp.float32)
        f2 = lax.dot_general(a2, wh[:, :_NHID], (((0,), (1,)), ((), ())),
                             preferred_element_type=jnp.float32)
        m_raw = f1 + jnp.max(f2)
        m = jnp.maximum(m_raw, _ALPHA * m_raw)
        l2e = 1.4426950408889634
        g1 = ((f1 - m) * l2e).astype(jnp.bfloat16)
        g2 = ((_ALPHA * f1 - m) * l2e).astype(jnp.bfloat16)
        f2b = (f2 * l2e).astype(jnp.bfloat16)
        f2c = (f2 * (_ALPHA * l2e)).astype(jnp.bfloat16)
        whbf = wh.astype(jnp.bfloat16)
        ck = 1024
        hp_aug = jnp.zeros((_BLK, _NHID + 1), jnp.float32)
        for c in range(_N // ck):
            sl = slice(c * ck, (c + 1) * ck)
            y = jnp.maximum(g1 + f2b[:, sl], g2 + f2c[:, sl])
            p = jnp.exp2(y) * adj[:, sl]
            hp_aug = hp_aug + jnp.dot(p, whbf[sl, :],
                                      preferred_element_type=jnp.float32)
        hp = hp_aug[:, :_NHID] / hp_aug[:, _NHID:]
        out_ref[:, h * _NHID:(h + 1) * _NHID] = jnp.where(
            hp > 0.0, hp, jnp.exp(hp) - 1.0)


@jax.jit
def kernel(x, adj, Ws, As):
    wh = pl.pallas_call(
        _wh_body,
        grid=(_NHEADS,),
        in_specs=[
            pl.BlockSpec((_N, _NFEAT), lambda h: (0, 0)),
            pl.BlockSpec((1, _NFEAT, _NHID), lambda h: (h, 0, 0)),
        ],
        out_specs=pl.BlockSpec((1, _N, _NHID + 1), lambda h: (h, 0, 0)),
        out_shape=jax.ShapeDtypeStruct((_NHEADS, _N, _NHID + 1), jnp.float32),
    )(x, Ws)

    out = pl.pallas_call(
        _gat_body,
        grid=(_N // _BLK,),
        in_specs=[
            pl.BlockSpec((_BLK, _N), lambda i: (i, 0)),
            pl.BlockSpec((_NHEADS, _N, _NHID + 1), lambda i: (0, 0, 0)),
            pl.BlockSpec((_NHEADS, 2 * _NHID, 1), lambda i: (0, 0, 0)),
        ],
        out_specs=pl.BlockSpec((_BLK, _NHEADS * _NHID), lambda i: (i, 0)),
        out_shape=jax.ShapeDtypeStruct((_N, _NHEADS * _NHID), jnp.float32),
    )(adj, wh, As)
    return out


# speedup vs baseline: 1.3707x; 1.0001x over previous
import functools

import jax
import jax.numpy as jnp
from jax import lax
from jax.experimental import pallas as pl
from jax.experimental.pallas import tpu as pltpu

_N = 4096
_NFEAT = 256
_NHID = 16
_NHEADS = 4
_ALPHA = 0.2
_BLK = 512


def _wh_body(x_ref, w_ref, wh_ref):
    wh = jnp.dot(x_ref[...], w_ref[0], preferred_element_type=jnp.float32)
    wh_ref[0] = jnp.concatenate(
        [wh, jnp.ones((_N, 1), jnp.float32)], axis=1)


def _gat_body(adj_ref, wh_ref, a_ref, out_ref):
    i = pl.program_id(0)
    adj = adj_ref[...].astype(jnp.bfloat16)
    for h in range(_NHEADS):
        wh = wh_ref[h]
        whb = wh_ref[h, pl.ds(i * _BLK, _BLK), :_NHID]
        a1 = a_ref[h, :_NHID, :]
        a2 = a_ref[h, _NHID:, :]
        f1 = lax.dot_general(whb, a1, (((1,), (0,)), ((), ())),
                             preferred_element_type=jnp.float32)
        f2 = lax.dot_general(a2, wh[:, :_NHID], (((0,), (1,)), ((), ())),
                             preferred_element_type=jnp.float32)
        m_raw = f1 + jnp.max(f2)
        m = jnp.maximum(m_raw, _ALPHA * m_raw)
        l2e = 1.4426950408889634
        g1 = ((f1 - m) * l2e).astype(jnp.bfloat16)
        g2 = ((_ALPHA * f1 - m) * l2e).astype(jnp.bfloat16)
        f2b = (f2 * l2e).astype(jnp.bfloat16)
        f2c = (f2 * (_ALPHA * l2e)).astype(jnp.bfloat16)
        whbf = wh.astype(jnp.bfloat16)
        ck = 1024
        hp_aug = jnp.zeros((_BLK, _NHID + 1), jnp.float32)
        for c in range(_N // ck):
            sl = slice(c * ck, (c + 1) * ck)
            y = jnp.maximum(g1 + f2b[:, sl], g2 + f2c[:, sl])
            p = jnp.exp2(y) * adj[:, sl]
            hp_aug = hp_aug + jnp.dot(p, whbf[sl, :],
                                      preferred_element_type=jnp.float32)
        hp = hp_aug[:, :_NHID] / hp_aug[:, _NHID:]
        out_ref[:, h * _NHID:(h + 1) * _NHID] = jnp.where(
            hp > 0.0, hp, jnp.exp(hp) - 1.0)


@jax.jit
def kernel(x, adj, Ws, As):
    wh = pl.pallas_call(
        _wh_body,
        grid=(_NHEADS,),
        in_specs=[
            pl.BlockSpec((_N, _NFEAT), lambda h: (0, 0)),
            pl.BlockSpec((1, _NFEAT, _NHID), lambda h: (h, 0, 0)),
        ],
        out_specs=pl.BlockSpec((1, _N, _NHID + 1), lambda h: (h, 0, 0)),
        out_shape=jax.ShapeDtypeStruct((_NHEADS, _N, _NHID + 1), jnp.float32),
    )(x, Ws)

    out = pl.pallas_call(
        _gat_body,
        grid=(_N // _BLK,),
        in_specs=[
            pl.BlockSpec((_BLK, _N), lambda i: (i, 0)),
            pl.BlockSpec((_NHEADS, _N, _NHID + 1), lambda i: (0, 0, 0)),
            pl.BlockSpec((_NHEADS, 2 * _NHID, 1), lambda i: (0, 0, 0)),
        ],
        out_specs=pl.BlockSpec((_BLK, _NHEADS * _NHID), lambda i: (i, 0)),
        out_shape=jax.ShapeDtypeStruct((_N, _NHEADS * _NHID), jnp.float32),
        compiler_params=pltpu.CompilerParams(
            dimension_semantics=("arbitrary",),
            vmem_limit_bytes=100 * 1024 * 1024,
        ),
    )(adj, wh, As)
    return out
